# baseline (speedup 1.0000x reference)
import functools

import jax
import jax.numpy as jnp
from jax import lax
from jax.experimental import pallas as pl
from jax.experimental.pallas import tpu as pltpu
from jax.experimental.pallas import tpu_sc as plsc

BATCH = 16384
EMBED = 32
HIDDEN = 64

_NC = 2
_NS = 16
_NW = _NC * _NS
_BPW = BATCH // _NW
_ICHUNK = 128
_NCHUNK = _BPW // _ICHUNK


def _sc_gather_body(uidx_hbm, iidx_hbm, utab_hbm, itab_hbm,
                    uemb_hbm, iemb_hbm,
                    uidx_v, iidx_v, urows_v, irows_v, sem):
    wid = lax.axis_index("s") * _NC + lax.axis_index("c")
    base = wid * _BPW
    pltpu.sync_copy(uidx_hbm.at[pl.ds(base, _BPW)], uidx_v)
    pltpu.sync_copy(iidx_hbm.at[pl.ds(base, _BPW)], iidx_v)
    copies = []
    for c in range(_NCHUNK):
        off = c * _ICHUNK
        copies.append(pltpu.async_copy(
            utab_hbm.at[uidx_v.at[pl.ds(off, _ICHUNK)]],
            urows_v.at[pl.ds(off, _ICHUNK)], sem))
        copies.append(pltpu.async_copy(
            itab_hbm.at[iidx_v.at[pl.ds(off, _ICHUNK)]],
            irows_v.at[pl.ds(off, _ICHUNK)], sem))
    for cp in copies:
        cp.wait()
    pltpu.sync_copy(urows_v, uemb_hbm.at[pl.ds(base, _BPW)])
    pltpu.sync_copy(irows_v, iemb_hbm.at[pl.ds(base, _BPW)])


_sc_gather = pl.kernel(
    _sc_gather_body,
    out_type=(
        jax.ShapeDtypeStruct((BATCH, EMBED), jnp.float32),
        jax.ShapeDtypeStruct((BATCH, EMBED), jnp.float32),
    ),
    mesh=plsc.VectorSubcoreMesh(core_axis_name="c", subcore_axis_name="s"),
    scratch_types=[
        pltpu.VMEM((_BPW,), jnp.int32),
        pltpu.VMEM((_BPW,), jnp.int32),
        pltpu.VMEM((_BPW, EMBED), jnp.float32),
        pltpu.VMEM((_BPW, EMBED), jnp.float32),
        pltpu.SemaphoreType.DMA,
    ],
    compiler_params=pltpu.CompilerParams(use_tc_tiling_on_sc=False),
)


_CHUNK = 2048


def _mlp_body(u_ref, v_ref, w1a_ref, w1b_ref, b1_ref, w2_ref, b2_ref, o_ref):
    h = jnp.dot(u_ref[...], w1a_ref[...],
                preferred_element_type=jnp.float32,
                precision=lax.Precision.HIGHEST)
    h = h + jnp.dot(v_ref[...], w1b_ref[...],
                    preferred_element_type=jnp.float32,
                    precision=lax.Precision.HIGHEST)
    h = jnp.maximum(h + b1_ref[...], 0.0)
    s = jnp.sum(h * w2_ref[...], axis=1) + b2_ref[0, 0]
    o_ref[...] = 1.0 / (1.0 + jnp.exp(-s))


@functools.partial(jax.jit, donate_argnums=())
def _run(user_indices, item_indices, user_table, item_table, W1, b1, W2, b2):
    uemb, iemb = _sc_gather(user_indices, item_indices, user_table, item_table)

    w1a = W1[:, :EMBED].T
    w1b = W1[:, EMBED:].T
    b1r = b1.reshape(1, HIDDEN)
    b2r = b2.reshape(1, 1)

    grid = BATCH // _CHUNK
    out = pl.pallas_call(
        _mlp_body,
        grid=(grid,),
        in_specs=[
            pl.BlockSpec((_CHUNK, EMBED), lambda i: (i, 0)),
            pl.BlockSpec((_CHUNK, EMBED), lambda i: (i, 0)),
            pl.BlockSpec((EMBED, HIDDEN), lambda i: (0, 0)),
            pl.BlockSpec((EMBED, HIDDEN), lambda i: (0, 0)),
            pl.BlockSpec((1, HIDDEN), lambda i: (0, 0)),
            pl.BlockSpec((1, HIDDEN), lambda i: (0, 0)),
            pl.BlockSpec((1, 1), lambda i: (0, 0)),
        ],
        out_specs=pl.BlockSpec((_CHUNK,), lambda i: (i,)),
        out_shape=jax.ShapeDtypeStruct((BATCH,), jnp.float32),
    )(uemb, iemb, w1a, w1b, b1r, W2, b2r)
    return out


def kernel(user_indices, item_indices, user_table, item_table, W1, b1, W2, b2):
    return _run(user_indices, item_indices, user_table, item_table,
                W1, b1, W2, b2)
